# baseline (speedup 1.0000x reference)
import jax
import jax.numpy as jnp
from jax import lax
from jax.experimental import pallas as pl
from jax.experimental.pallas import tpu as pltpu
from jax.experimental.pallas import tpu_sc as plsc

_NB = 16384
_NS = 50
_D = 32
_NW = 32
_BPW = _NB // _NW
_Q = 128
_NQ = _BPW // _Q
_HALF = 256


def _emb_body(xT, w128, out,
              idx0, idx1, rows0, rows1, offs0, offs1,
              gb0, gb1, gb2, gb3, ov0, ov1,
              isem0, isem1, gsem0, gsem1, gsem2, gsem3, osem0, osem1):
    idxv = (idx0, idx1)
    rows = (rows0, rows1)
    offs = (offs0, offs1)
    gbuf = (gb0, gb1, gb2, gb3)
    outv = (ov0, ov1)
    isem = (isem0, isem1)
    gsem = (gsem0, gsem1, gsem2, gsem3)
    osem = (osem0, osem1)

    wid = lax.axis_index("s") * 2 + lax.axis_index("c")
    b0 = wid * _BPW
    iota = lax.iota(jnp.int32, 16)

    def fire_idx(s, p):
        pltpu.async_copy(xT.at[s, pl.ds(b0, _BPW)], idxv[p], isem[p])

    def wait_idx(p):
        pltpu.make_async_copy(xT.at[0, pl.ds(0, _BPW)], idxv[p],
                              isem[p]).wait()

    def prep(p):
        def body(k, c):
            e = idxv[p][pl.ds(16 * k, 16)]
            rows[p][pl.ds(16 * k, 16)] = e >> 2
            offs[p][pl.ds(16 * k, 16)] = (e & 3) << 5
            return c
        lax.fori_loop(0, _BPW // 16, body, 0)

    def fire_gather(p, q):
        pltpu.async_copy(w128.at[rows[p].at[pl.ds(q * _Q, _Q)]],
                         gbuf[q], gsem[q])

    def wait_gather(p, q):
        pltpu.make_async_copy(w128.at[rows[p].at[pl.ds(0, _Q)]],
                              gbuf[q], gsem[q]).wait()

    def extract(p, q):
        j0 = (q % 2) * _Q
        def body(k, c):
            t = iota + 16 * k
            off = offs[p][pl.ds(q * _Q + 16 * k, 16)]
            for d in range(_D):
                outv[q // 2][d, pl.ds(j0 + 16 * k, 16)] = plsc.load_gather(
                    gbuf[q], [t, off + d])
            return c
        lax.fori_loop(0, _Q // 16, body, 0)

    def fire_store(s, j):
        pltpu.async_copy(outv[j], out.at[s, :, pl.ds(b0 + j * _HALF, _HALF)],
                         osem[j])

    def wait_store(j):
        pltpu.make_async_copy(outv[j], out.at[0, :, pl.ds(0, _HALF)],
                              osem[j]).wait()

    fire_idx(0, 0)
    wait_idx(0)
    prep(0)
    for q in range(_NQ):
        fire_gather(0, q)
    fire_idx(1, 1)

    def one_s(s, p, first, last):
        p1 = 1 - p
        if not last:
            wait_idx(p1)
            prep(p1)
        for q in range(_NQ):
            j = q // 2
            if q % 2 == 0 and not first:
                wait_store(j)
            wait_gather(p, q)
            extract(p, q)
            if not last:
                fire_gather(p1, q)
            if q % 2 == 1:
                fire_store(s, j)
        if not last:
            @pl.when(s + 2 < _NS)
            def _():
                fire_idx(s + 2, p)

    one_s(0, 0, True, False)

    def pair(m, carry):
        one_s(2 * m + 1, 1, False, False)
        one_s(2 * m + 2, 0, False, False)
        return carry

    lax.fori_loop(0, (_NS - 2) // 2, pair, 0)
    one_s(_NS - 1, 1, False, True)
    wait_store(0)
    wait_store(1)


def kernel(x, weight):
    xT = x.T
    w128 = weight.reshape(250000, 128)
    mesh = plsc.VectorSubcoreMesh(core_axis_name="c", subcore_axis_name="s")
    outT = pl.kernel(
        _emb_body,
        out_type=jax.ShapeDtypeStruct((_NS, _D, _NB), jnp.float32),
        mesh=mesh,
        scratch_types=[
            pltpu.VMEM((_BPW,), jnp.int32),
            pltpu.VMEM((_BPW,), jnp.int32),
            pltpu.VMEM((_BPW,), jnp.int32),
            pltpu.VMEM((_BPW,), jnp.int32),
            pltpu.VMEM((_BPW,), jnp.int32),
            pltpu.VMEM((_BPW,), jnp.int32),
            pltpu.VMEM((_Q, 128), jnp.float32),
            pltpu.VMEM((_Q, 128), jnp.float32),
            pltpu.VMEM((_Q, 128), jnp.float32),
            pltpu.VMEM((_Q, 128), jnp.float32),
            pltpu.VMEM((_D, _HALF), jnp.float32),
            pltpu.VMEM((_D, _HALF), jnp.float32),
            pltpu.SemaphoreType.DMA,
            pltpu.SemaphoreType.DMA,
            pltpu.SemaphoreType.DMA,
            pltpu.SemaphoreType.DMA,
            pltpu.SemaphoreType.DMA,
            pltpu.SemaphoreType.DMA,
            pltpu.SemaphoreType.DMA,
            pltpu.SemaphoreType.DMA,
        ],
        compiler_params=pltpu.CompilerParams(needs_layout_passes=False),
    )(xT, w128)
    return outT.transpose(2, 0, 1)


# speedup vs baseline: 1.4479x; 1.4479x over previous
import jax
import jax.numpy as jnp
from jax import lax
from jax.experimental import pallas as pl
from jax.experimental.pallas import tpu as pltpu
from jax.experimental.pallas import tpu_sc as plsc

_NB = 16384
_NS = 50
_D = 32
_NW = 32
_BPW = _NB // _NW
_Q = 128
_NQ = _BPW // _Q
_HALF = 256


def _emb_body(xT, w128, out,
              idx0, idx1, rows0, rows1, offs0, offs1,
              gb0, gb1, gb2, gb3, ov0, ov1,
              isem0, isem1, gsem0, gsem1, gsem2, gsem3, osem0, osem1):
    idxv = (idx0, idx1)
    rows = (rows0, rows1)
    offs = (offs0, offs1)
    gbuf = (gb0, gb1, gb2, gb3)
    outv = (ov0, ov1)
    isem = (isem0, isem1)
    gsem = (gsem0, gsem1, gsem2, gsem3)
    osem = (osem0, osem1)

    wid = lax.axis_index("s") * 2 + lax.axis_index("c")
    b0 = wid * _BPW
    iota = lax.iota(jnp.int32, 16)

    def fire_idx(s, p):
        pltpu.async_copy(xT.at[s, pl.ds(b0, _BPW)], idxv[p], isem[p])

    def wait_idx(p):
        pltpu.make_async_copy(xT.at[0, pl.ds(0, _BPW)], idxv[p],
                              isem[p]).wait()

    def prep(p):
        def body(k, c):
            e = idxv[p][pl.ds(16 * k, 16)]
            rows[p][pl.ds(16 * k, 16)] = e >> 2
            offs[p][pl.ds(16 * k, 16)] = (e & 3) << 5
            return c
        lax.fori_loop(0, _BPW // 16, body, 0)

    def fire_gather(p, q):
        pltpu.async_copy(w128.at[rows[p].at[pl.ds(q * _Q, _Q)]],
                         gbuf[q], gsem[q])

    def wait_gather(p, q):
        pltpu.make_async_copy(w128.at[rows[p].at[pl.ds(0, _Q)]],
                              gbuf[q], gsem[q]).wait()

    def extract(p, q):
        j0 = (q % 2) * _Q
        def body(k, c):
            t = iota + 16 * k
            t_out = t + j0
            off = offs[p][pl.ds(q * _Q + 16 * k, 16)]
            for d in range(_D):
                f = (iota + d) & 31
                v = plsc.load_gather(gbuf[q], [t, off + f])
                plsc.store_scatter(outv[q // 2], [f, t_out], v)
            return c
        lax.fori_loop(0, _Q // 16, body, 0)

    def fire_store(s, j):
        pltpu.async_copy(outv[j], out.at[s, :, pl.ds(b0 + j * _HALF, _HALF)],
                         osem[j])

    def wait_store(j):
        pltpu.make_async_copy(outv[j], out.at[0, :, pl.ds(0, _HALF)],
                              osem[j]).wait()

    fire_idx(0, 0)
    wait_idx(0)
    prep(0)
    for q in range(_NQ):
        fire_gather(0, q)
    fire_idx(1, 1)

    def one_s(s, p, first, last):
        p1 = 1 - p
        if not last:
            wait_idx(p1)
            prep(p1)
        for q in range(_NQ):
            j = q // 2
            if q % 2 == 0 and not first:
                wait_store(j)
            wait_gather(p, q)
            extract(p, q)
            if not last:
                fire_gather(p1, q)
            if q % 2 == 1:
                fire_store(s, j)
        if not last:
            @pl.when(s + 2 < _NS)
            def _():
                fire_idx(s + 2, p)

    one_s(0, 0, True, False)

    def pair(m, carry):
        one_s(2 * m + 1, 1, False, False)
        one_s(2 * m + 2, 0, False, False)
        return carry

    lax.fori_loop(0, (_NS - 2) // 2, pair, 0)
    one_s(_NS - 1, 1, False, True)
    wait_store(0)
    wait_store(1)


def kernel(x, weight):
    xT = x.T
    w128 = weight.reshape(250000, 128)
    mesh = plsc.VectorSubcoreMesh(core_axis_name="c", subcore_axis_name="s")
    outT = pl.kernel(
        _emb_body,
        out_type=jax.ShapeDtypeStruct((_NS, _D, _NB), jnp.float32),
        mesh=mesh,
        scratch_types=[
            pltpu.VMEM((_BPW,), jnp.int32),
            pltpu.VMEM((_BPW,), jnp.int32),
            pltpu.VMEM((_BPW,), jnp.int32),
            pltpu.VMEM((_BPW,), jnp.int32),
            pltpu.VMEM((_BPW,), jnp.int32),
            pltpu.VMEM((_BPW,), jnp.int32),
            pltpu.VMEM((_Q, 128), jnp.float32),
            pltpu.VMEM((_Q, 128), jnp.float32),
            pltpu.VMEM((_Q, 128), jnp.float32),
            pltpu.VMEM((_Q, 128), jnp.float32),
            pltpu.VMEM((_D, _HALF), jnp.float32),
            pltpu.VMEM((_D, _HALF), jnp.float32),
            pltpu.SemaphoreType.DMA,
            pltpu.SemaphoreType.DMA,
            pltpu.SemaphoreType.DMA,
            pltpu.SemaphoreType.DMA,
            pltpu.SemaphoreType.DMA,
            pltpu.SemaphoreType.DMA,
            pltpu.SemaphoreType.DMA,
            pltpu.SemaphoreType.DMA,
        ],
        compiler_params=pltpu.CompilerParams(needs_layout_passes=False),
    )(xT, w128)
    return outT.transpose(2, 0, 1)
